# Initial kernel scaffold; baseline (speedup 1.0000x reference)
import jax
import jax.numpy as jnp
from jax.experimental import pallas as pl


def kernel(coarse_logits, fine_features, point_coords, W1, b1, W2, b2, W3, b3, Wf, bf):
    raise NotImplementedError("write your pallas kernel here")


# speedup vs baseline: 4.8062x; 4.8062x over previous
import functools

import jax
import jax.numpy as jnp
from jax import lax
from jax.experimental import pallas as pl
from jax.experimental.pallas import tpu as pltpu
from jax.experimental.pallas import tpu_sc as plsc


def _floor_frac(v):
    t = v.astype(jnp.int32)
    f0 = jnp.where(v < t.astype(jnp.float32), t - 1, t)
    fr = v - f0.astype(jnp.float32)
    return f0, fr


def _z_kernel(f_ref, w_ref, z_ref):
    z_ref[0] = lax.dot_general(
        f_ref[0], w_ref[...], (((0,), (1,)), ((), ())),
        preferred_element_type=jnp.float32)


def _mlp_kernel(h_ref, w2_ref, b2_ref, w3_ref, b3_ref, wf_ref, bf_ref, o_ref):
    h = h_ref[...]
    h = jnp.maximum(
        lax.dot_general(h, w2_ref[...], (((1,), (1,)), ((), ())),
                        preferred_element_type=jnp.float32) + b2_ref[...], 0.0)
    h = jnp.maximum(
        lax.dot_general(h, w3_ref[...], (((1,), (1,)), ((), ())),
                        preferred_element_type=jnp.float32) + b3_ref[...], 0.0)
    o_ref[...] = (jnp.sum(h * wf_ref[...], axis=1, keepdims=True)
                  + bf_ref[0, 0])


def kernel(coarse_logits, fine_features, point_coords, W1, b1, W2, b2, W3, b3, Wf, bf):
    B, Cout, Hc, Wc = coarse_logits.shape
    _, Cin, Hf, Wfp = fine_features.shape
    P = point_coords.shape[1]
    FC = W2.shape[0]
    NPIX = Hf * Wfp
    NCPIX = Hc * Wc
    NPTS = B * P

    fine = fine_features.reshape(B, Cin, NPIX)
    W1f = W1[:, :Cin]
    w1c = W1[:, Cin]
    xs = point_coords[..., 0].reshape(NPTS)
    ys = point_coords[..., 1].reshape(NPTS)
    coarse = coarse_logits.reshape(B, NCPIX)

    PBLK = 2048
    z = pl.pallas_call(
        _z_kernel,
        grid=(B, NPIX // PBLK),
        in_specs=[
            pl.BlockSpec((1, Cin, PBLK), lambda b, j: (b, 0, j)),
            pl.BlockSpec((FC, Cin), lambda b, j: (0, 0)),
        ],
        out_specs=pl.BlockSpec((1, PBLK, FC), lambda b, j: (b, j, 0)),
        out_shape=jax.ShapeDtypeStruct((B, NPIX, FC), jnp.float32),
    )(fine, W1f)
    z2d = z.reshape(B * NPIX, FC)

    info = plsc.get_sparse_core_info()
    NC, NS = info.num_cores, info.num_subcores
    NW = NC * NS
    PT = NPTS // NW
    CHUNK = 64
    NCHUNK = PT // CHUNK
    mesh = plsc.VectorSubcoreMesh(core_axis_name="c", subcore_axis_name="s")

    @functools.partial(
        pl.kernel,
        out_type=jax.ShapeDtypeStruct((NPTS, FC), jnp.float32),
        mesh=mesh,
        compiler_params=pltpu.CompilerParams(needs_layout_passes=False),
        scratch_types=[
            pltpu.VMEM((CHUNK,), jnp.float32),
            pltpu.VMEM((CHUNK,), jnp.float32),
            pltpu.VMEM((4, CHUNK), jnp.int32),
            pltpu.VMEM((CHUNK,), jnp.float32),
            pltpu.VMEM((CHUNK,), jnp.float32),
            pltpu.VMEM((CHUNK,), jnp.float32),
            pltpu.VMEM((CHUNK,), jnp.float32),
            pltpu.VMEM((CHUNK,), jnp.float32),
            pltpu.VMEM((4 * CHUNK, FC), jnp.float32),
            pltpu.VMEM((CHUNK, FC), jnp.float32),
            pltpu.VMEM((NCPIX,), jnp.float32),
            pltpu.VMEM((FC,), jnp.float32),
            pltpu.VMEM((FC,), jnp.float32),
            pltpu.SemaphoreType.DMA,
        ],
    )
    def _sc_stage(z_hbm, xs_hbm, ys_hbm, coarse_hbm, w1c_hbm, b1_hbm, h1_hbm,
                  xs_v, ys_v, idx_v, w00_v, w01_v, w10_v, w11_v, cs_v,
                  rows_v, h1_v, coarse_v, w1c_v, b1_v, sem):
        cid = lax.axis_index("c")
        sid = lax.axis_index("s")
        wid = sid * NC + cid
        bidx = wid // (NW // B)
        base = wid * PT
        zbase = bidx * NPIX

        pltpu.sync_copy(coarse_hbm.at[bidx], coarse_v)
        pltpu.sync_copy(w1c_hbm, w1c_v)
        pltpu.sync_copy(b1_hbm, b1_v)

        def chunk_body(ci, carry):
            cbase = base + ci * CHUNK
            pltpu.sync_copy(xs_hbm.at[pl.ds(cbase, CHUNK)], xs_v)
            pltpu.sync_copy(ys_hbm.at[pl.ds(cbase, CHUNK)], ys_v)

            for g in range(CHUNK // 16):
                s = pl.ds(g * 16, 16)
                x = xs_v[s]
                y = ys_v[s]
                ix = x * Wfp - 0.5
                iy = y * Hf - 0.5
                ix0, fx = _floor_frac(ix)
                iy0, fy = _floor_frac(iy)
                zx0 = jnp.where(ix0 >= 0, 1.0 - fx, 0.0)
                zx1 = jnp.where(ix0 <= Wfp - 2, fx, 0.0)
                zy0 = jnp.where(iy0 >= 0, 1.0 - fy, 0.0)
                zy1 = jnp.where(iy0 <= Hf - 2, fy, 0.0)
                cx0 = jnp.maximum(ix0, 0)
                cx1 = jnp.minimum(ix0 + 1, Wfp - 1)
                cy0 = jnp.maximum(iy0, 0)
                cy1 = jnp.minimum(iy0 + 1, Hf - 1)
                idx_v[0, s] = zbase + cy0 * Wfp + cx0
                idx_v[1, s] = zbase + cy0 * Wfp + cx1
                idx_v[2, s] = zbase + cy1 * Wfp + cx0
                idx_v[3, s] = zbase + cy1 * Wfp + cx1
                w00_v[s] = zx0 * zy0
                w01_v[s] = zx1 * zy0
                w10_v[s] = zx0 * zy1
                w11_v[s] = zx1 * zy1
                jx = x * Wc - 0.5
                jy = y * Hc - 0.5
                jx0, gx = _floor_frac(jx)
                jy0, gy = _floor_frac(jy)
                ax0 = jnp.where(jx0 >= 0, 1.0 - gx, 0.0)
                ax1 = jnp.where(jx0 <= Wc - 2, gx, 0.0)
                ay0 = jnp.where(jy0 >= 0, 1.0 - gy, 0.0)
                ay1 = jnp.where(jy0 <= Hc - 2, gy, 0.0)
                px0 = jnp.maximum(jx0, 0)
                px1 = jnp.minimum(jx0 + 1, Wc - 1)
                py0 = jnp.maximum(jy0, 0)
                py1 = jnp.minimum(jy0 + 1, Hc - 1)
                c00 = plsc.load_gather(coarse_v, [py0 * Wc + px0])
                c01 = plsc.load_gather(coarse_v, [py0 * Wc + px1])
                c10 = plsc.load_gather(coarse_v, [py1 * Wc + px0])
                c11 = plsc.load_gather(coarse_v, [py1 * Wc + px1])
                cs_v[s] = (c00 * (ax0 * ay0) + c01 * (ax1 * ay0)
                           + c10 * (ax0 * ay1) + c11 * (ax1 * ay1))

            cps = [pltpu.async_copy(z_hbm.at[idx_v.at[k]],
                                    rows_v.at[pl.ds(k * CHUNK, CHUNK)], sem)
                   for k in range(4)]
            for cp in cps:
                cp.wait()

            def pt_body(p, c2):
                pv = jnp.full((16,), 0, jnp.int32) + p
                w00 = plsc.load_gather(w00_v, [pv])
                w01 = plsc.load_gather(w01_v, [pv])
                w10 = plsc.load_gather(w10_v, [pv])
                w11 = plsc.load_gather(w11_v, [pv])
                csv = plsc.load_gather(cs_v, [pv])
                for c in range(FC // 16):
                    cslice = pl.ds(c * 16, 16)
                    acc = rows_v[p, cslice] * w00
                    acc = acc + rows_v[CHUNK + p, cslice] * w01
                    acc = acc + rows_v[2 * CHUNK + p, cslice] * w10
                    acc = acc + rows_v[3 * CHUNK + p, cslice] * w11
                    acc = acc + csv * w1c_v[cslice]
                    acc = acc + b1_v[cslice]
                    h1_v[p, cslice] = jnp.maximum(acc, 0.0)
                return c2

            lax.fori_loop(0, CHUNK, pt_body, 0)
            pltpu.sync_copy(h1_v, h1_hbm.at[pl.ds(cbase, CHUNK)])
            return carry

        lax.fori_loop(0, NCHUNK, chunk_body, 0)

    h1 = _sc_stage(z2d, xs, ys, coarse, w1c, b1)

    PB = 2048
    out = pl.pallas_call(
        _mlp_kernel,
        grid=(NPTS // PB,),
        in_specs=[
            pl.BlockSpec((PB, FC), lambda i: (i, 0)),
            pl.BlockSpec((FC, FC), lambda i: (0, 0)),
            pl.BlockSpec((1, FC), lambda i: (0, 0)),
            pl.BlockSpec((FC, FC), lambda i: (0, 0)),
            pl.BlockSpec((1, FC), lambda i: (0, 0)),
            pl.BlockSpec((Cout, FC), lambda i: (0, 0)),
            pl.BlockSpec((1, Cout), lambda i: (0, 0)),
        ],
        out_specs=pl.BlockSpec((PB, Cout), lambda i: (i, 0)),
        out_shape=jax.ShapeDtypeStruct((NPTS, Cout), jnp.float32),
    )(h1, W2, b2.reshape(1, FC), W3, b3.reshape(1, FC),
      Wf, bf.reshape(1, Cout))

    return out.reshape(B, P, Cout).transpose(0, 2, 1)


# speedup vs baseline: 5.9226x; 1.2323x over previous
import functools

import jax
import jax.numpy as jnp
from jax import lax
from jax.experimental import pallas as pl
from jax.experimental.pallas import tpu as pltpu
from jax.experimental.pallas import tpu_sc as plsc


def _floor_frac(v):
    t = v.astype(jnp.int32)
    f0 = jnp.where(v < t.astype(jnp.float32), t - 1, t)
    fr = v - f0.astype(jnp.float32)
    return f0, fr


def _z_kernel(f_ref, w_ref, z_ref):
    rb = f_ref.shape[2]
    wfp = f_ref.shape[3]
    for r in range(rb):
        z_ref[pl.ds(r * wfp, wfp), :] = lax.dot_general(
            f_ref[0, :, r, :], w_ref[...], (((0,), (1,)), ((), ())),
            preferred_element_type=jnp.float32)


def _mlp_kernel(h_ref, w2_ref, b2_ref, w3_ref, b3_ref, wf_ref, bf_ref, o_ref):
    h = h_ref[...]
    h = jnp.maximum(
        lax.dot_general(h, w2_ref[...], (((1,), (1,)), ((), ())),
                        preferred_element_type=jnp.float32) + b2_ref[...], 0.0)
    h = jnp.maximum(
        lax.dot_general(h, w3_ref[...], (((1,), (1,)), ((), ())),
                        preferred_element_type=jnp.float32) + b3_ref[...], 0.0)
    o_ref[...] = (jnp.sum(h * wf_ref[...], axis=1, keepdims=True)
                  + bf_ref[0, 0])


def kernel(coarse_logits, fine_features, point_coords, W1, b1, W2, b2, W3, b3, Wf, bf):
    B, Cout, Hc, Wc = coarse_logits.shape
    _, Cin, Hf, Wfp = fine_features.shape
    P = point_coords.shape[1]
    FC = W2.shape[0]
    NPIX = Hf * Wfp
    NCPIX = Hc * Wc
    NPTS = B * P

    W1f = W1[:, :Cin]
    w1c = W1[:, Cin]
    xs = point_coords[..., 0].reshape(NPTS)
    ys = point_coords[..., 1].reshape(NPTS)
    coarse = coarse_logits.reshape(B, NCPIX)

    RB = 16
    z2d = pl.pallas_call(
        _z_kernel,
        grid=(B, Hf // RB),
        in_specs=[
            pl.BlockSpec((1, Cin, RB, Wfp), lambda b, r: (b, 0, r, 0)),
            pl.BlockSpec((FC, Cin), lambda b, r: (0, 0)),
        ],
        out_specs=pl.BlockSpec((RB * Wfp, FC),
                               lambda b, r: (b * (Hf // RB) + r, 0)),
        out_shape=jax.ShapeDtypeStruct((B * NPIX, FC), jnp.float32),
    )(fine_features, W1f)

    info = plsc.get_sparse_core_info()
    NC, NS = info.num_cores, info.num_subcores
    NW = NC * NS
    PT = NPTS // NW
    CHUNK = 64
    NCHUNK = PT // CHUNK
    mesh = plsc.VectorSubcoreMesh(core_axis_name="c", subcore_axis_name="s")

    @functools.partial(
        pl.kernel,
        out_type=jax.ShapeDtypeStruct((NPTS, FC), jnp.float32),
        mesh=mesh,
        compiler_params=pltpu.CompilerParams(needs_layout_passes=False),
        scratch_types=[
            pltpu.VMEM((CHUNK,), jnp.float32),
            pltpu.VMEM((CHUNK,), jnp.float32),
            pltpu.VMEM((4, CHUNK), jnp.int32),
            pltpu.VMEM((CHUNK,), jnp.float32),
            pltpu.VMEM((CHUNK,), jnp.float32),
            pltpu.VMEM((CHUNK,), jnp.float32),
            pltpu.VMEM((CHUNK,), jnp.float32),
            pltpu.VMEM((CHUNK,), jnp.float32),
            pltpu.VMEM((4 * CHUNK, FC), jnp.float32),
            pltpu.VMEM((CHUNK, FC), jnp.float32),
            pltpu.VMEM((NCPIX,), jnp.float32),
            pltpu.VMEM((FC,), jnp.float32),
            pltpu.VMEM((FC,), jnp.float32),
            pltpu.SemaphoreType.DMA,
        ],
    )
    def _sc_stage(z_hbm, xs_hbm, ys_hbm, coarse_hbm, w1c_hbm, b1_hbm, h1_hbm,
                  xs_v, ys_v, idx_v, w00_v, w01_v, w10_v, w11_v, cs_v,
                  rows_v, h1_v, coarse_v, w1c_v, b1_v, sem):
        cid = lax.axis_index("c")
        sid = lax.axis_index("s")
        wid = sid * NC + cid
        bidx = wid // (NW // B)
        base = wid * PT
        zbase = bidx * NPIX

        pltpu.sync_copy(coarse_hbm.at[bidx], coarse_v)
        pltpu.sync_copy(w1c_hbm, w1c_v)
        pltpu.sync_copy(b1_hbm, b1_v)

        def chunk_body(ci, carry):
            cbase = base + ci * CHUNK
            pltpu.sync_copy(xs_hbm.at[pl.ds(cbase, CHUNK)], xs_v)
            pltpu.sync_copy(ys_hbm.at[pl.ds(cbase, CHUNK)], ys_v)

            for g in range(CHUNK // 16):
                s = pl.ds(g * 16, 16)
                x = xs_v[s]
                y = ys_v[s]
                ix = x * Wfp - 0.5
                iy = y * Hf - 0.5
                ix0, fx = _floor_frac(ix)
                iy0, fy = _floor_frac(iy)
                zx0 = jnp.where(ix0 >= 0, 1.0 - fx, 0.0)
                zx1 = jnp.where(ix0 <= Wfp - 2, fx, 0.0)
                zy0 = jnp.where(iy0 >= 0, 1.0 - fy, 0.0)
                zy1 = jnp.where(iy0 <= Hf - 2, fy, 0.0)
                cx0 = jnp.maximum(ix0, 0)
                cx1 = jnp.minimum(ix0 + 1, Wfp - 1)
                cy0 = jnp.maximum(iy0, 0)
                cy1 = jnp.minimum(iy0 + 1, Hf - 1)
                idx_v[0, s] = zbase + cy0 * Wfp + cx0
                idx_v[1, s] = zbase + cy0 * Wfp + cx1
                idx_v[2, s] = zbase + cy1 * Wfp + cx0
                idx_v[3, s] = zbase + cy1 * Wfp + cx1
                w00_v[s] = zx0 * zy0
                w01_v[s] = zx1 * zy0
                w10_v[s] = zx0 * zy1
                w11_v[s] = zx1 * zy1
                jx = x * Wc - 0.5
                jy = y * Hc - 0.5
                jx0, gx = _floor_frac(jx)
                jy0, gy = _floor_frac(jy)
                ax0 = jnp.where(jx0 >= 0, 1.0 - gx, 0.0)
                ax1 = jnp.where(jx0 <= Wc - 2, gx, 0.0)
                ay0 = jnp.where(jy0 >= 0, 1.0 - gy, 0.0)
                ay1 = jnp.where(jy0 <= Hc - 2, gy, 0.0)
                px0 = jnp.maximum(jx0, 0)
                px1 = jnp.minimum(jx0 + 1, Wc - 1)
                py0 = jnp.maximum(jy0, 0)
                py1 = jnp.minimum(jy0 + 1, Hc - 1)
                c00 = plsc.load_gather(coarse_v, [py0 * Wc + px0])
                c01 = plsc.load_gather(coarse_v, [py0 * Wc + px1])
                c10 = plsc.load_gather(coarse_v, [py1 * Wc + px0])
                c11 = plsc.load_gather(coarse_v, [py1 * Wc + px1])
                cs_v[s] = (c00 * (ax0 * ay0) + c01 * (ax1 * ay0)
                           + c10 * (ax0 * ay1) + c11 * (ax1 * ay1))

            cps = [pltpu.async_copy(z_hbm.at[idx_v.at[k]],
                                    rows_v.at[pl.ds(k * CHUNK, CHUNK)], sem)
                   for k in range(4)]
            for cp in cps:
                cp.wait()

            def pt_body(p, c2):
                pv = jnp.full((16,), 0, jnp.int32) + p
                w00 = plsc.load_gather(w00_v, [pv])
                w01 = plsc.load_gather(w01_v, [pv])
                w10 = plsc.load_gather(w10_v, [pv])
                w11 = plsc.load_gather(w11_v, [pv])
                csv = plsc.load_gather(cs_v, [pv])
                for c in range(FC // 16):
                    cslice = pl.ds(c * 16, 16)
                    acc = rows_v[p, cslice] * w00
                    acc = acc + rows_v[CHUNK + p, cslice] * w01
                    acc = acc + rows_v[2 * CHUNK + p, cslice] * w10
                    acc = acc + rows_v[3 * CHUNK + p, cslice] * w11
                    acc = acc + csv * w1c_v[cslice]
                    acc = acc + b1_v[cslice]
                    h1_v[p, cslice] = jnp.maximum(acc, 0.0)
                return c2

            lax.fori_loop(0, CHUNK, pt_body, 0)
            pltpu.sync_copy(h1_v, h1_hbm.at[pl.ds(cbase, CHUNK)])
            return carry

        lax.fori_loop(0, NCHUNK, chunk_body, 0)

    h1 = _sc_stage(z2d, xs, ys, coarse, w1c, b1)

    PB = 2048
    out = pl.pallas_call(
        _mlp_kernel,
        grid=(NPTS // PB,),
        in_specs=[
            pl.BlockSpec((PB, FC), lambda i: (i, 0)),
            pl.BlockSpec((FC, FC), lambda i: (0, 0)),
            pl.BlockSpec((1, FC), lambda i: (0, 0)),
            pl.BlockSpec((FC, FC), lambda i: (0, 0)),
            pl.BlockSpec((1, FC), lambda i: (0, 0)),
            pl.BlockSpec((Cout, FC), lambda i: (0, 0)),
            pl.BlockSpec((1, Cout), lambda i: (0, 0)),
        ],
        out_specs=pl.BlockSpec((PB, Cout), lambda i: (i, 0)),
        out_shape=jax.ShapeDtypeStruct((NPTS, Cout), jnp.float32),
    )(h1, W2, b2.reshape(1, FC), W3, b3.reshape(1, FC),
      Wf, bf.reshape(1, Cout))

    return out.reshape(B, P, Cout).transpose(0, 2, 1)


# speedup vs baseline: 7.4200x; 1.2528x over previous
import functools

import jax
import jax.numpy as jnp
from jax import lax
from jax.experimental import pallas as pl
from jax.experimental.pallas import tpu as pltpu
from jax.experimental.pallas import tpu_sc as plsc


def _floor_frac(v):
    t = v.astype(jnp.int32)
    f0 = jnp.where(v < t.astype(jnp.float32), t - 1, t)
    fr = v - f0.astype(jnp.float32)
    return f0, fr


def _z_kernel(f_ref, w_ref, z_ref):
    rb = f_ref.shape[2]
    wfp = f_ref.shape[3]
    for r in range(rb):
        z_ref[pl.ds(r * wfp, wfp), :] = lax.dot_general(
            f_ref[0, :, r, :], w_ref[...], (((0,), (1,)), ((), ())),
            preferred_element_type=jnp.float32)


def _mlp_kernel(h_ref, w2_ref, b2_ref, w3_ref, b3_ref, wf_ref, bf_ref, o_ref):
    h = h_ref[...]
    h = jnp.maximum(
        lax.dot_general(h, w2_ref[...], (((1,), (1,)), ((), ())),
                        preferred_element_type=jnp.float32) + b2_ref[...], 0.0)
    h = jnp.maximum(
        lax.dot_general(h, w3_ref[...], (((1,), (1,)), ((), ())),
                        preferred_element_type=jnp.float32) + b3_ref[...], 0.0)
    o_ref[...] = (jnp.sum(h * wf_ref[...], axis=1, keepdims=True)
                  + bf_ref[0, 0])


def kernel(coarse_logits, fine_features, point_coords, W1, b1, W2, b2, W3, b3, Wf, bf):
    B, Cout, Hc, Wc = coarse_logits.shape
    _, Cin, Hf, Wfp = fine_features.shape
    P = point_coords.shape[1]
    FC = W2.shape[0]
    NPIX = Hf * Wfp
    NCPIX = Hc * Wc
    NPTS = B * P

    W1f = W1[:, :Cin]
    w1c = W1[:, Cin]
    xs = point_coords[..., 0].reshape(NPTS)
    ys = point_coords[..., 1].reshape(NPTS)
    coarse = coarse_logits.reshape(B, NCPIX)

    RB = 16
    z2d = pl.pallas_call(
        _z_kernel,
        grid=(B, Hf // RB),
        in_specs=[
            pl.BlockSpec((1, Cin, RB, Wfp), lambda b, r: (b, 0, r, 0)),
            pl.BlockSpec((FC, Cin), lambda b, r: (0, 0)),
        ],
        out_specs=pl.BlockSpec((RB * Wfp, FC),
                               lambda b, r: (b * (Hf // RB) + r, 0)),
        out_shape=jax.ShapeDtypeStruct((B * NPIX, FC), jnp.float32),
    )(fine_features, W1f)

    info = plsc.get_sparse_core_info()
    NC, NS = info.num_cores, info.num_subcores
    NW = NC * NS
    PT = NPTS // NW
    CHUNK = 32
    NCHUNK = PT // CHUNK
    mesh = plsc.VectorSubcoreMesh(core_axis_name="c", subcore_axis_name="s")

    @functools.partial(
        pl.kernel,
        out_type=jax.ShapeDtypeStruct((NPTS, FC), jnp.float32),
        mesh=mesh,
        compiler_params=pltpu.CompilerParams(needs_layout_passes=False),
        scratch_types=[
            pltpu.VMEM((PT,), jnp.float32),
            pltpu.VMEM((PT,), jnp.float32),
            pltpu.VMEM((4, PT), jnp.int32),
            pltpu.VMEM((5, PT), jnp.float32),
            pltpu.VMEM((4 * CHUNK, FC), jnp.float32),
            pltpu.VMEM((4 * CHUNK, FC), jnp.float32),
            pltpu.VMEM((CHUNK, FC), jnp.float32),
            pltpu.VMEM((CHUNK, FC), jnp.float32),
            pltpu.VMEM((NCPIX,), jnp.float32),
            pltpu.VMEM((FC,), jnp.float32),
            pltpu.VMEM((FC,), jnp.float32),
            pltpu.SemaphoreType.DMA,
            pltpu.SemaphoreType.DMA,
            pltpu.SemaphoreType.DMA,
            pltpu.SemaphoreType.DMA,
        ],
    )
    def _sc_stage(z_hbm, xs_hbm, ys_hbm, coarse_hbm, w1c_hbm, b1_hbm, h1_hbm,
                  xs_v, ys_v, idx_a, wgt_a, rows0, rows1, h1a, h1b,
                  coarse_v, w1c_v, b1_v, gsem0, gsem1, wsem0, wsem1):
        cid = lax.axis_index("c")
        sid = lax.axis_index("s")
        wid = sid * NC + cid
        bidx = wid // (NW // B)
        base = wid * PT
        zbase = bidx * NPIX

        pltpu.sync_copy(coarse_hbm.at[bidx], coarse_v)
        pltpu.sync_copy(w1c_hbm, w1c_v)
        pltpu.sync_copy(b1_hbm, b1_v)
        pltpu.sync_copy(xs_hbm.at[pl.ds(base, PT)], xs_v)
        pltpu.sync_copy(ys_hbm.at[pl.ds(base, PT)], ys_v)

        def wgt_body(gi, carry):
            s = pl.ds(gi * 16, 16)
            x = xs_v[s]
            y = ys_v[s]
            ix = x * Wfp - 0.5
            iy = y * Hf - 0.5
            ix0, fx = _floor_frac(ix)
            iy0, fy = _floor_frac(iy)
            zx0 = jnp.where(ix0 >= 0, 1.0 - fx, 0.0)
            zx1 = jnp.where(ix0 <= Wfp - 2, fx, 0.0)
            zy0 = jnp.where(iy0 >= 0, 1.0 - fy, 0.0)
            zy1 = jnp.where(iy0 <= Hf - 2, fy, 0.0)
            cx0 = jnp.maximum(ix0, 0)
            cx1 = jnp.minimum(ix0 + 1, Wfp - 1)
            cy0 = jnp.maximum(iy0, 0)
            cy1 = jnp.minimum(iy0 + 1, Hf - 1)
            idx_a[0, s] = zbase + cy0 * Wfp + cx0
            idx_a[1, s] = zbase + cy0 * Wfp + cx1
            idx_a[2, s] = zbase + cy1 * Wfp + cx0
            idx_a[3, s] = zbase + cy1 * Wfp + cx1
            wgt_a[0, s] = zx0 * zy0
            wgt_a[1, s] = zx1 * zy0
            wgt_a[2, s] = zx0 * zy1
            wgt_a[3, s] = zx1 * zy1
            jx = x * Wc - 0.5
            jy = y * Hc - 0.5
            jx0, gx = _floor_frac(jx)
            jy0, gy = _floor_frac(jy)
            ax0 = jnp.where(jx0 >= 0, 1.0 - gx, 0.0)
            ax1 = jnp.where(jx0 <= Wc - 2, gx, 0.0)
            ay0 = jnp.where(jy0 >= 0, 1.0 - gy, 0.0)
            ay1 = jnp.where(jy0 <= Hc - 2, gy, 0.0)
            px0 = jnp.maximum(jx0, 0)
            px1 = jnp.minimum(jx0 + 1, Wc - 1)
            py0 = jnp.maximum(jy0, 0)
            py1 = jnp.minimum(jy0 + 1, Hc - 1)
            c00 = plsc.load_gather(coarse_v, [py0 * Wc + px0])
            c01 = plsc.load_gather(coarse_v, [py0 * Wc + px1])
            c10 = plsc.load_gather(coarse_v, [py1 * Wc + px0])
            c11 = plsc.load_gather(coarse_v, [py1 * Wc + px1])
            wgt_a[4, s] = (c00 * (ax0 * ay0) + c01 * (ax1 * ay0)
                           + c10 * (ax0 * ay1) + c11 * (ax1 * ay1))
            return carry

        lax.fori_loop(0, PT // 16, wgt_body, 0)

        def gather_descs(ci, rows_v, sem):
            cb = ci * CHUNK
            return [pltpu.make_async_copy(
                z_hbm.at[idx_a.at[k, pl.ds(cb, CHUNK)]],
                rows_v.at[pl.ds(k * CHUNK, CHUNK)], sem)
                for k in range(4)]

        def start_gathers(ci, rows_v, sem):
            for cp in gather_descs(ci, rows_v, sem):
                cp.start()

        def wait_gathers(ci, rows_v, sem):
            for cp in gather_descs(ci, rows_v, sem):
                cp.wait()

        def write_desc(ci, h1_v, sem):
            return pltpu.make_async_copy(
                h1_v, h1_hbm.at[pl.ds(base + ci * CHUNK, CHUNK)], sem)

        def combine(ci, rows_v, h1_v):
            pbase = ci * CHUNK

            def pt_body(p, c2):
                pv = jnp.full((16,), 0, jnp.int32) + (pbase + p)
                w00 = plsc.load_gather(wgt_a, [jnp.full((16,), 0, jnp.int32), pv])
                w01 = plsc.load_gather(wgt_a, [jnp.full((16,), 1, jnp.int32), pv])
                w10 = plsc.load_gather(wgt_a, [jnp.full((16,), 2, jnp.int32), pv])
                w11 = plsc.load_gather(wgt_a, [jnp.full((16,), 3, jnp.int32), pv])
                csv = plsc.load_gather(wgt_a, [jnp.full((16,), 4, jnp.int32), pv])
                for c in range(FC // 16):
                    cslice = pl.ds(c * 16, 16)
                    acc = rows_v[p, cslice] * w00
                    acc = acc + rows_v[CHUNK + p, cslice] * w01
                    acc = acc + rows_v[2 * CHUNK + p, cslice] * w10
                    acc = acc + rows_v[3 * CHUNK + p, cslice] * w11
                    acc = acc + csv * w1c_v[cslice]
                    acc = acc + b1_v[cslice]
                    h1_v[p, cslice] = jnp.maximum(acc, 0.0)
                return c2

            lax.fori_loop(0, CHUNK, pt_body, 0)

        start_gathers(0, rows0, gsem0)

        def loop_body(i, carry):
            ci0 = 2 * i
            start_gathers(ci0 + 1, rows1, gsem1)
            wait_gathers(ci0, rows0, gsem0)

            @pl.when(i > 0)
            def _():
                write_desc(ci0 - 2, h1a, wsem0).wait()

            combine(ci0, rows0, h1a)
            write_desc(ci0, h1a, wsem0).start()

            @pl.when(i < NCHUNK // 2 - 1)
            def _():
                start_gathers(ci0 + 2, rows0, gsem0)

            wait_gathers(ci0 + 1, rows1, gsem1)

            @pl.when(i > 0)
            def _():
                write_desc(ci0 - 1, h1b, wsem1).wait()

            combine(ci0 + 1, rows1, h1b)
            write_desc(ci0 + 1, h1b, wsem1).start()
            return carry

        lax.fori_loop(0, NCHUNK // 2, loop_body, 0)
        write_desc(NCHUNK - 2, h1a, wsem0).wait()
        write_desc(NCHUNK - 1, h1b, wsem1).wait()

    h1 = _sc_stage(z2d, xs, ys, coarse, w1c, b1)

    PB = 2048
    out = pl.pallas_call(
        _mlp_kernel,
        grid=(NPTS // PB,),
        in_specs=[
            pl.BlockSpec((PB, FC), lambda i: (i, 0)),
            pl.BlockSpec((FC, FC), lambda i: (0, 0)),
            pl.BlockSpec((1, FC), lambda i: (0, 0)),
            pl.BlockSpec((FC, FC), lambda i: (0, 0)),
            pl.BlockSpec((1, FC), lambda i: (0, 0)),
            pl.BlockSpec((Cout, FC), lambda i: (0, 0)),
            pl.BlockSpec((1, Cout), lambda i: (0, 0)),
        ],
        out_specs=pl.BlockSpec((PB, Cout), lambda i: (i, 0)),
        out_shape=jax.ShapeDtypeStruct((NPTS, Cout), jnp.float32),
    )(h1, W2, b2.reshape(1, FC), W3, b3.reshape(1, FC),
      Wf, bf.reshape(1, Cout))

    return out.reshape(B, P, Cout).transpose(0, 2, 1)
